# baseline (speedup 1.0000x reference)
import functools

import numpy as np
import jax
import jax.numpy as jnp
from jax import lax
from jax.experimental import pallas as pl
from jax.experimental.pallas import tpu as pltpu
from jax.experimental.pallas import tpu_sc as plsc

B, S = 4, 4096
HID = 1024
NGRAM_MAX = 3
NHEAD = 8
NUM_HEADS = (NGRAM_MAX - 1) * NHEAD
D = 64
VOCAB_PER_HEAD = 80800
EH = NUM_HEADS * D
PAD_ID = 2
BS = B * S
TOTAL_ROWS = BS * NUM_HEADS
TOTAL_N = NUM_HEADS * VOCAB_PER_HEAD
HALF_N = TOTAL_N // 2
HSTREAM = TOTAL_ROWS // 2

_rng = np.random.default_rng(10007)
_mults = (_rng.integers(1, 2 ** 20, size=(NGRAM_MAX,)).astype(np.int32) * 2 + 1)
M0, M1, M2 = int(_mults[0]), int(_mults[1]), int(_mults[2])


RB = 3200


def _repack_body(t_ref, o_ref):
    h = pl.program_id(1)

    @pl.when(h == 0)
    def _lo():
        o_ref[:, 0:D] = t_ref[...]

    @pl.when(h == 1)
    def _hi():
        o_ref[:, D:2 * D] = t_ref[...]


def _repack(table):
    nsteps = HALF_N // RB
    return pl.pallas_call(
        _repack_body,
        grid=(nsteps, 2),
        in_specs=[
            pl.BlockSpec((RB, D), lambda i, h, n=nsteps: (h * n + i, 0)),
        ],
        out_specs=pl.BlockSpec((RB, 2 * D), lambda i, h: (i, 0)),
        out_shape=jax.ShapeDtypeStruct((HALF_N, 2 * D), jnp.float32),
    )(table)


NW = 32
BS_PER_TILE = BS // NW
ROWS_PER_TILE = BS_PER_TILE * NHEAD
CHUNK_ROWS = 256
NCHUNK = ROWS_PER_TILE // CHUNK_ROWS

_sc_mesh = plsc.VectorSubcoreMesh(core_axis_name="c", subcore_axis_name="s")


@functools.partial(
    pl.kernel,
    mesh=_sc_mesh,
    compiler_params=pltpu.CompilerParams(needs_layout_passes=False,
                                         use_tc_tiling_on_sc=False),
    out_type=(jax.ShapeDtypeStruct((HSTREAM, 2 * D), jnp.float32),
              jax.ShapeDtypeStruct((HSTREAM, 2 * D), jnp.float32),
              jax.ShapeDtypeStruct((HSTREAM,), jnp.int32),
              jax.ShapeDtypeStruct((HSTREAM,), jnp.int32)),
    scratch_types=[
        pltpu.VMEM((16 + BS_PER_TILE,), jnp.int32),
        pltpu.VMEM((ROWS_PER_TILE,), jnp.int32),
        pltpu.VMEM((ROWS_PER_TILE,), jnp.int32),
        pltpu.VMEM((ROWS_PER_TILE,), jnp.int32),
        pltpu.VMEM((ROWS_PER_TILE,), jnp.int32),
        pltpu.VMEM((2, CHUNK_ROWS, 2 * D), jnp.float32),
        pltpu.SemaphoreType.DMA,
    ],
)
def _sc_hash_gather(tok_hbm, t2_hbm, outa_hbm, outb_hbm, para_hbm, parb_hbm,
                    tok_v, idxa_v, idxb_v, para_v, parb_v, rows_v, gsem):
    wid = lax.axis_index("s") * 2 + lax.axis_index("c")
    base = wid * BS_PER_TILE
    pltpu.sync_copy(tok_hbm.at[pl.ds(base, BS_PER_TILE)],
                    tok_v.at[pl.ds(16, BS_PER_TILE)])

    @pl.when(wid > 0)
    def _load_ctx():
        pltpu.sync_copy(tok_hbm.at[pl.ds(base - 16, 16)], tok_v.at[pl.ds(0, 16)])

    iota = lax.iota(jnp.int32, 16)
    s0 = (wid % (S // BS_PER_TILE)) * BS_PER_TILE
    seq_start = 1 - jnp.minimum(s0, 1)

    def hash_block(i, carry):
        t0 = tok_v[pl.ds(16 + i * 16, 16)]
        t1 = tok_v[pl.ds(15 + i * 16, 16)]
        t2 = tok_v[pl.ds(14 + i * 16, 16)]
        first = seq_start * (1 - jnp.minimum(i, 1))
        m1 = 1 - first * (1 - jnp.minimum(iota, 1))
        m2 = 1 - first * (1 - jnp.minimum(iota, 2) // 2)
        t1 = t1 * m1 + PAD_ID * (1 - m1)
        t2 = t2 * m2 + PAD_ID * (1 - m2)
        h2 = t0 * M0 + t1 * M1
        h3 = h2 + t2 * M2
        q0 = i * (16 * NHEAD) + iota * NHEAD
        for h in range(NUM_HEADS):
            hb = h2 if h < NHEAD else h3
            head = h % NHEAD
            hv = hb * (2 * head + 1) + head * 10007
            r = lax.rem(hv, VOCAB_PER_HEAD)
            r = r + (lax.shift_right_arithmetic(r, 31) & VOCAB_PER_HEAD)
            rid = r + h * VOCAB_PER_HEAD
            half = 1 + lax.shift_right_arithmetic(rid - HALF_N, 31)
            pos = q0 + h // 2
            if h % 2 == 0:
                plsc.store_scatter(idxa_v, [pos], rid - half * HALF_N)
                plsc.store_scatter(para_v, [pos], half)
            else:
                plsc.store_scatter(idxb_v, [pos], rid - half * HALF_N)
                plsc.store_scatter(parb_v, [pos], half)
        return carry

    lax.fori_loop(0, BS_PER_TILE // 16, hash_block, 0)

    rbase = wid * ROWS_PER_TILE
    pltpu.sync_copy(para_v, para_hbm.at[pl.ds(rbase, ROWS_PER_TILE)])
    pltpu.sync_copy(parb_v, parb_hbm.at[pl.ds(rbase, ROWS_PER_TILE)])

    plan = []
    for j in range(NCHUNK):
        plan.append((idxa_v, outa_hbm, j))
        plan.append((idxb_v, outb_hbm, j))
    cps = [None, None]
    prev = [None, None]
    for k, (idx_ref, out_ref, j) in enumerate(plan):
        p = k & 1
        cps[p] = pltpu.async_copy(
            t2_hbm.at[idx_ref.at[pl.ds(j * CHUNK_ROWS, CHUNK_ROWS)]],
            rows_v.at[p], gsem)
        if k > 0:
            po_ref, pj = prev[1 - p]
            cps[1 - p].wait()
            pltpu.sync_copy(
                rows_v.at[1 - p],
                po_ref.at[pl.ds(rbase + pj * CHUNK_ROWS, CHUNK_ROWS)])
        prev[p] = (out_ref, j)
    lastp = (len(plan) - 1) & 1
    po_ref, pj = prev[lastp]
    cps[lastp].wait()
    pltpu.sync_copy(rows_v.at[lastp],
                    po_ref.at[pl.ds(rbase + pj * CHUNK_ROWS, CHUNK_ROWS)])


BM = 512
BR = BM * NHEAD
NPROJ = 5


def _tc_proj_body(xa_ref, xb_ref, pa_ref, pb_ref,
                  wv_ref, bv_ref, wk_ref, bk_ref, nw_ref, o_ref):
    rawa = xa_ref[...]
    rawb = xb_ref[...]
    para = lax.broadcast_in_dim(pa_ref[...], (BR, D), (0,))
    parb = lax.broadcast_in_dim(pb_ref[...], (BR, D), (0,))
    a64 = jnp.where(para == 1, rawa[:, D:2 * D], rawa[:, 0:D])
    b64 = jnp.where(parb == 1, rawb[:, D:2 * D], rawb[:, 0:D])
    x128 = jnp.concatenate([a64, b64], axis=1)
    x = x128.reshape(BM, EH).astype(jnp.bfloat16)
    y = lax.dot_general(x, wv_ref[...], (((1,), (1,)), ((), ())),
                        preferred_element_type=jnp.float32)
    o_ref[0] = y + bv_ref[...][None, :]
    for c in range(NPROJ - 1):
        y = lax.dot_general(x, wk_ref[c], (((1,), (1,)), ((), ())),
                            preferred_element_type=jnp.float32)
        y = y + bk_ref[c][None, :]
        var = jnp.mean(y * y, axis=-1, keepdims=True)
        o_ref[c + 1] = y * lax.rsqrt(var + 1e-6) * nw_ref[c][None, :]


def _tc_proj(pa, pb, para, parb, Wv, bv, Wk, bk, norm_w):
    return pl.pallas_call(
        _tc_proj_body,
        grid=(BS // BM,),
        in_specs=[
            pl.BlockSpec((BR, 2 * D), lambda i: (i, 0)),
            pl.BlockSpec((BR, 2 * D), lambda i: (i, 0)),
            pl.BlockSpec((BR,), lambda i: (i,)),
            pl.BlockSpec((BR,), lambda i: (i,)),
            pl.BlockSpec((HID, EH), lambda i: (0, 0)),
            pl.BlockSpec((HID,), lambda i: (0,)),
            pl.BlockSpec((NPROJ - 1, HID, EH), lambda i: (0, 0, 0)),
            pl.BlockSpec((NPROJ - 1, HID), lambda i: (0, 0)),
            pl.BlockSpec((NPROJ - 1, HID), lambda i: (0, 0)),
        ],
        out_specs=pl.BlockSpec((NPROJ, BM, HID), lambda i: (0, i, 0)),
        out_shape=jax.ShapeDtypeStruct((NPROJ, BS, HID), jnp.float32),
    )(pa, pb, para, parb, Wv, bv, Wk, bk, norm_w)


def kernel(input_ids, table, Wv, bv, Wk, bk, norm_w):
    t2 = _repack(table)
    pa, pb, para, parb = _sc_hash_gather(input_ids.reshape(-1), t2)
    out = _tc_proj(pa, pb, para, parb, Wv.astype(jnp.bfloat16), bv,
                   Wk.astype(jnp.bfloat16), bk, norm_w)
    return out.reshape(NPROJ, B, S, HID)


# speedup vs baseline: 1.0190x; 1.0190x over previous
import functools

import numpy as np
import jax
import jax.numpy as jnp
from jax import lax
from jax.experimental import pallas as pl
from jax.experimental.pallas import tpu as pltpu
from jax.experimental.pallas import tpu_sc as plsc

B, S = 4, 4096
HID = 1024
NGRAM_MAX = 3
NHEAD = 8
NUM_HEADS = (NGRAM_MAX - 1) * NHEAD
D = 64
VOCAB_PER_HEAD = 80800
EH = NUM_HEADS * D
PAD_ID = 2
BS = B * S
TOTAL_ROWS = BS * NUM_HEADS
TOTAL_N = NUM_HEADS * VOCAB_PER_HEAD
HALF_N = TOTAL_N // 2
HSTREAM = TOTAL_ROWS // 2

_rng = np.random.default_rng(10007)
_mults = (_rng.integers(1, 2 ** 20, size=(NGRAM_MAX,)).astype(np.int32) * 2 + 1)
M0, M1, M2 = int(_mults[0]), int(_mults[1]), int(_mults[2])


RB = 3200


def _repack(table):
    return jnp.concatenate([table[:HALF_N], table[HALF_N:]], axis=1)


NW = 32
BS_PER_TILE = BS // NW
ROWS_PER_TILE = BS_PER_TILE * NHEAD
CHUNK_ROWS = 256
NCHUNK = ROWS_PER_TILE // CHUNK_ROWS

_sc_mesh = plsc.VectorSubcoreMesh(core_axis_name="c", subcore_axis_name="s")


@functools.partial(
    pl.kernel,
    mesh=_sc_mesh,
    compiler_params=pltpu.CompilerParams(needs_layout_passes=False,
                                         use_tc_tiling_on_sc=False),
    out_type=(jax.ShapeDtypeStruct((HSTREAM, 2 * D), jnp.float32),
              jax.ShapeDtypeStruct((HSTREAM, 2 * D), jnp.float32),
              jax.ShapeDtypeStruct((HSTREAM,), jnp.int32),
              jax.ShapeDtypeStruct((HSTREAM,), jnp.int32)),
    scratch_types=[
        pltpu.VMEM((16 + BS_PER_TILE,), jnp.int32),
        pltpu.VMEM((ROWS_PER_TILE,), jnp.int32),
        pltpu.VMEM((ROWS_PER_TILE,), jnp.int32),
        pltpu.VMEM((ROWS_PER_TILE,), jnp.int32),
        pltpu.VMEM((ROWS_PER_TILE,), jnp.int32),
        pltpu.VMEM((2, CHUNK_ROWS, 2 * D), jnp.float32),
        pltpu.SemaphoreType.DMA,
    ],
)
def _sc_hash_gather(tok_hbm, t2_hbm, outa_hbm, outb_hbm, para_hbm, parb_hbm,
                    tok_v, idxa_v, idxb_v, para_v, parb_v, rows_v, gsem):
    wid = lax.axis_index("s") * 2 + lax.axis_index("c")
    base = wid * BS_PER_TILE
    pltpu.sync_copy(tok_hbm.at[pl.ds(base, BS_PER_TILE)],
                    tok_v.at[pl.ds(16, BS_PER_TILE)])

    @pl.when(wid > 0)
    def _load_ctx():
        pltpu.sync_copy(tok_hbm.at[pl.ds(base - 16, 16)], tok_v.at[pl.ds(0, 16)])

    iota = lax.iota(jnp.int32, 16)
    s0 = (wid % (S // BS_PER_TILE)) * BS_PER_TILE
    seq_start = 1 - jnp.minimum(s0, 1)

    def hash_block(i, carry):
        t0 = tok_v[pl.ds(16 + i * 16, 16)]
        t1 = tok_v[pl.ds(15 + i * 16, 16)]
        t2 = tok_v[pl.ds(14 + i * 16, 16)]
        first = seq_start * (1 - jnp.minimum(i, 1))
        m1 = 1 - first * (1 - jnp.minimum(iota, 1))
        m2 = 1 - first * (1 - jnp.minimum(iota, 2) // 2)
        t1 = t1 * m1 + PAD_ID * (1 - m1)
        t2 = t2 * m2 + PAD_ID * (1 - m2)
        h2 = t0 * M0 + t1 * M1
        h3 = h2 + t2 * M2
        q0 = i * (16 * NHEAD) + iota * NHEAD
        for h in range(NUM_HEADS):
            hb = h2 if h < NHEAD else h3
            head = h % NHEAD
            hv = hb * (2 * head + 1) + head * 10007
            r = lax.rem(hv, VOCAB_PER_HEAD)
            r = r + (lax.shift_right_arithmetic(r, 31) & VOCAB_PER_HEAD)
            rid = r + h * VOCAB_PER_HEAD
            half = 1 + lax.shift_right_arithmetic(rid - HALF_N, 31)
            pos = q0 + h // 2
            if h % 2 == 0:
                plsc.store_scatter(idxa_v, [pos], rid - half * HALF_N)
                plsc.store_scatter(para_v, [pos], half)
            else:
                plsc.store_scatter(idxb_v, [pos], rid - half * HALF_N)
                plsc.store_scatter(parb_v, [pos], half)
        return carry

    lax.fori_loop(0, BS_PER_TILE // 16, hash_block, 0)

    rbase = wid * ROWS_PER_TILE
    pltpu.sync_copy(para_v, para_hbm.at[pl.ds(rbase, ROWS_PER_TILE)])
    pltpu.sync_copy(parb_v, parb_hbm.at[pl.ds(rbase, ROWS_PER_TILE)])

    plan = []
    for j in range(NCHUNK):
        plan.append((idxa_v, outa_hbm, j))
        plan.append((idxb_v, outb_hbm, j))
    cps = [None, None]
    prev = [None, None]
    for k, (idx_ref, out_ref, j) in enumerate(plan):
        p = k & 1
        cps[p] = pltpu.async_copy(
            t2_hbm.at[idx_ref.at[pl.ds(j * CHUNK_ROWS, CHUNK_ROWS)]],
            rows_v.at[p], gsem)
        if k > 0:
            po_ref, pj = prev[1 - p]
            cps[1 - p].wait()
            pltpu.sync_copy(
                rows_v.at[1 - p],
                po_ref.at[pl.ds(rbase + pj * CHUNK_ROWS, CHUNK_ROWS)])
        prev[p] = (out_ref, j)
    lastp = (len(plan) - 1) & 1
    po_ref, pj = prev[lastp]
    cps[lastp].wait()
    pltpu.sync_copy(rows_v.at[lastp],
                    po_ref.at[pl.ds(rbase + pj * CHUNK_ROWS, CHUNK_ROWS)])


BM = 512
BR = BM * NHEAD
NPROJ = 5


def _tc_proj_body(xa_ref, xb_ref, pa_ref, pb_ref,
                  wv_ref, bv_ref, wk_ref, bk_ref, nw_ref, o_ref):
    rawa = xa_ref[...]
    rawb = xb_ref[...]
    para = lax.broadcast_in_dim(pa_ref[...], (BR, D), (0,))
    parb = lax.broadcast_in_dim(pb_ref[...], (BR, D), (0,))
    a64 = jnp.where(para == 1, rawa[:, D:2 * D], rawa[:, 0:D])
    b64 = jnp.where(parb == 1, rawb[:, D:2 * D], rawb[:, 0:D])
    x128 = jnp.concatenate([a64, b64], axis=1)
    x = x128.reshape(BM, EH).astype(jnp.bfloat16)
    y = lax.dot_general(x, wv_ref[...], (((1,), (1,)), ((), ())),
                        preferred_element_type=jnp.float32)
    o_ref[0] = y + bv_ref[...][None, :]
    for c in range(NPROJ - 1):
        y = lax.dot_general(x, wk_ref[c], (((1,), (1,)), ((), ())),
                            preferred_element_type=jnp.float32)
        y = y + bk_ref[c][None, :]
        var = jnp.mean(y * y, axis=-1, keepdims=True)
        o_ref[c + 1] = y * lax.rsqrt(var + 1e-6) * nw_ref[c][None, :]


def _tc_proj(pa, pb, para, parb, Wv, bv, Wk, bk, norm_w):
    return pl.pallas_call(
        _tc_proj_body,
        grid=(BS // BM,),
        in_specs=[
            pl.BlockSpec((BR, 2 * D), lambda i: (i, 0)),
            pl.BlockSpec((BR, 2 * D), lambda i: (i, 0)),
            pl.BlockSpec((BR,), lambda i: (i,)),
            pl.BlockSpec((BR,), lambda i: (i,)),
            pl.BlockSpec((HID, EH), lambda i: (0, 0)),
            pl.BlockSpec((HID,), lambda i: (0,)),
            pl.BlockSpec((NPROJ - 1, HID, EH), lambda i: (0, 0, 0)),
            pl.BlockSpec((NPROJ - 1, HID), lambda i: (0, 0)),
            pl.BlockSpec((NPROJ - 1, HID), lambda i: (0, 0)),
        ],
        out_specs=pl.BlockSpec((NPROJ, BM, HID), lambda i: (0, i, 0)),
        out_shape=jax.ShapeDtypeStruct((NPROJ, BS, HID), jnp.float32),
    )(pa, pb, para, parb, Wv, bv, Wk, bk, norm_w)


def kernel(input_ids, table, Wv, bv, Wk, bk, norm_w):
    t2 = _repack(table)
    pa, pb, para, parb = _sc_hash_gather(input_ids.reshape(-1), t2)
    out = _tc_proj(pa, pb, para, parb, Wv.astype(jnp.bfloat16), bv,
                   Wk.astype(jnp.bfloat16), bk, norm_w)
    return out.reshape(NPROJ, B, S, HID)


# speedup vs baseline: 1.3494x; 1.3242x over previous
import functools

import numpy as np
import jax
import jax.numpy as jnp
from jax import lax
from jax.experimental import pallas as pl
from jax.experimental.pallas import tpu as pltpu
from jax.experimental.pallas import tpu_sc as plsc

B, S = 4, 4096
HID = 1024
NGRAM_MAX = 3
NHEAD = 8
NUM_HEADS = (NGRAM_MAX - 1) * NHEAD
D = 64
VOCAB_PER_HEAD = 80800
EH = NUM_HEADS * D
PAD_ID = 2
BS = B * S
TOTAL_ROWS = BS * NUM_HEADS

_rng = np.random.default_rng(10007)
_mults = (_rng.integers(1, 2 ** 20, size=(NGRAM_MAX,)).astype(np.int32) * 2 + 1)
M0, M1, M2 = int(_mults[0]), int(_mults[1]), int(_mults[2])


NW = 32
BS_PER_TILE = BS // NW
ROWS_PER_TILE = BS_PER_TILE * NUM_HEADS
CHUNK_BS = 32
CHUNK_ROWS = CHUNK_BS * NUM_HEADS
NCHUNK = BS_PER_TILE // CHUNK_BS

_sc_mesh = plsc.VectorSubcoreMesh(core_axis_name="c", subcore_axis_name="s")


@functools.partial(
    pl.kernel,
    mesh=_sc_mesh,
    compiler_params=pltpu.CompilerParams(needs_layout_passes=False,
                                         use_tc_tiling_on_sc=False),
    out_type=jax.ShapeDtypeStruct((TOTAL_ROWS, D), jnp.float32),
    scratch_types=[
        pltpu.VMEM((16 + BS_PER_TILE,), jnp.int32),
        pltpu.VMEM((ROWS_PER_TILE,), jnp.int32),
        pltpu.VMEM((2, CHUNK_ROWS, D), jnp.float32),
        pltpu.SemaphoreType.DMA,
    ],
)
def _sc_hash_gather(tok_hbm, table_hbm, out_hbm, tok_v, idx_v, rows_v, gsem):
    wid = lax.axis_index("s") * 2 + lax.axis_index("c")
    base = wid * BS_PER_TILE
    s0 = (wid % (S // BS_PER_TILE)) * BS_PER_TILE
    pltpu.sync_copy(tok_hbm.at[pl.ds(base, BS_PER_TILE)],
                    tok_v.at[pl.ds(16, BS_PER_TILE)])

    @pl.when(wid > 0)
    def _load_ctx():
        pltpu.sync_copy(tok_hbm.at[pl.ds(base - 16, 16)], tok_v.at[pl.ds(0, 16)])

    iota = lax.iota(jnp.int32, 16)
    seq_start = 1 - jnp.minimum(s0, 1)

    def hash_block(i, carry):
        t0 = tok_v[pl.ds(16 + i * 16, 16)]
        t1 = tok_v[pl.ds(15 + i * 16, 16)]
        t2 = tok_v[pl.ds(14 + i * 16, 16)]
        first = seq_start * (1 - jnp.minimum(i, 1))
        m1 = 1 - first * (1 - jnp.minimum(iota, 1))
        m2 = 1 - first * (1 - jnp.minimum(iota, 2) // 2)
        t1 = t1 * m1 + PAD_ID * (1 - m1)
        t2 = t2 * m2 + PAD_ID * (1 - m2)
        h2 = t0 * M0 + t1 * M1
        h3 = h2 + t2 * M2
        q0 = i * (16 * NUM_HEADS) + iota * NUM_HEADS
        for h in range(NUM_HEADS):
            hb = h2 if h < NHEAD else h3
            head = h % NHEAD
            hv = hb * (2 * head + 1) + head * 10007
            r = lax.rem(hv, VOCAB_PER_HEAD)
            r = r + (lax.shift_right_arithmetic(r, 31) & VOCAB_PER_HEAD)
            plsc.store_scatter(idx_v, [q0 + h], r + h * VOCAB_PER_HEAD)
        return carry

    lax.fori_loop(0, BS_PER_TILE // 16, hash_block, 0)

    rbase = wid * ROWS_PER_TILE
    cps = [None, None]
    for j in range(NCHUNK):
        p = j & 1
        cps[p] = pltpu.async_copy(
            table_hbm.at[idx_v.at[pl.ds(j * CHUNK_ROWS, CHUNK_ROWS)]],
            rows_v.at[p], gsem)
        if j > 0:
            cps[1 - p].wait()
            pltpu.sync_copy(
                rows_v.at[1 - p],
                out_hbm.at[pl.ds(rbase + (j - 1) * CHUNK_ROWS, CHUNK_ROWS)])
    cps[(NCHUNK - 1) & 1].wait()
    pltpu.sync_copy(
        rows_v.at[(NCHUNK - 1) & 1],
        out_hbm.at[pl.ds(rbase + (NCHUNK - 1) * CHUNK_ROWS, CHUNK_ROWS)])


BM = 512
NPROJ = 5


def _tc_proj_body(x_ref, wv_ref, bv_ref, wk_ref, bk_ref, nw_ref, o_ref):
    x = x_ref[...].reshape(BM, EH).astype(jnp.bfloat16)
    y = lax.dot_general(x, wv_ref[...], (((1,), (1,)), ((), ())),
                        preferred_element_type=jnp.float32)
    o_ref[0] = y + bv_ref[...][None, :]
    for c in range(NPROJ - 1):
        y = lax.dot_general(x, wk_ref[c], (((1,), (1,)), ((), ())),
                            preferred_element_type=jnp.float32)
        y = y + bk_ref[c][None, :]
        var = jnp.mean(y * y, axis=-1, keepdims=True)
        o_ref[c + 1] = y * lax.rsqrt(var + 1e-6) * nw_ref[c][None, :]


def _tc_proj(emb, Wv, bv, Wk, bk, norm_w):
    return pl.pallas_call(
        _tc_proj_body,
        grid=(BS // BM,),
        in_specs=[
            pl.BlockSpec((BM * 8, 128), lambda i: (i, 0)),
            pl.BlockSpec((HID, EH), lambda i: (0, 0)),
            pl.BlockSpec((HID,), lambda i: (0,)),
            pl.BlockSpec((NPROJ - 1, HID, EH), lambda i: (0, 0, 0)),
            pl.BlockSpec((NPROJ - 1, HID), lambda i: (0, 0)),
            pl.BlockSpec((NPROJ - 1, HID), lambda i: (0, 0)),
        ],
        out_specs=pl.BlockSpec((NPROJ, BM, HID), lambda i: (0, i, 0)),
        out_shape=jax.ShapeDtypeStruct((NPROJ, BS, HID), jnp.float32),
    )(emb, Wv, bv, Wk, bk, norm_w)


def kernel(input_ids, table, Wv, bv, Wk, bk, norm_w):
    emb = _sc_hash_gather(input_ids.reshape(-1), table)
    emb = emb.reshape(BS * EH // 128, 128)
    out = _tc_proj(emb, Wv.astype(jnp.bfloat16), bv,
                   Wk.astype(jnp.bfloat16), bk, norm_w)
    return out.reshape(NPROJ, B, S, HID)
